# baseline (speedup 1.0000x reference)
import jax
import jax.numpy as jnp
from jax import lax
from jax.experimental import pallas as pl
from jax.experimental.pallas import tpu as pltpu

B, P = 32, 16800
LANES = 128
CH = 2048
NC = 9
PW = NC * CH
NSTEP = 4 * NC
GRID = NSTEP + 1
NEG_POS_RATIO = 7
BOX_WEIGHT = 2.0
INT32_MIN = -2147483648


def _smooth_l1(x):
    a = jnp.abs(x)
    return jnp.where(a < 1.0, 0.5 * x * x, a - 0.5)


def _mbl_kernel(ct, cp, lp, lt, dp, dt, out, key_s, nl_s, accf, acci):
    step = pl.program_id(0)

    @pl.when(step == 0)
    def _init():
        accf[0] = 0.0
        accf[1] = 0.0
        accf[2] = 0.0
        acci[0] = 0
        acci[1] = 0

    @pl.when(step < NSTEP)
    def _accumulate():
        t = step // NC
        cb = step % NC

        lane = lax.broadcasted_iota(jnp.int32, (8, CH), 1)
        valid = cb * CH + lane < P

        labels = ct[...]
        pos = (labels > 0) & valid
        neg = (labels == 0) & valid

        x0 = cp[:, 0, :]
        x1 = cp[:, 1, :]
        d = x1 - x0
        z = jnp.where(pos, -d, d)
        spl = jnp.maximum(z, 0.0) + jnp.log(1.0 + jnp.exp(-jnp.abs(z)))

        accf[0] += jnp.sum(jnp.where(pos, spl, 0.0))
        acci[0] += jnp.sum(pos, dtype=jnp.int32)
        acci[1] += jnp.sum(neg, dtype=jnp.int32)

        bits = lax.bitcast_convert_type(d, jnp.int32)
        key = jnp.where(bits >= 0, bits, INT32_MIN - bits)
        key = jnp.where(neg, key, INT32_MIN)
        key_s[pl.ds(t * 8, 8), pl.ds(cb * CH, CH)] = key
        nl_s[pl.ds(t * 8, 8), pl.ds(cb * CH, CH)] = jnp.where(neg, spl, 0.0)

        t4 = _smooth_l1(lp[:, 0, :] - lt[:, 0, :])
        for c in range(1, 4):
            t4 += _smooth_l1(lp[:, c, :] - lt[:, c, :])
        accf[1] += jnp.sum(jnp.where(pos, t4, 0.0))

        t10 = _smooth_l1(dp[0] - dt[0])
        badc = (dt[0] == -1.0).astype(jnp.int32)
        for c in range(1, 10):
            t10 += _smooth_l1(dp[c] - dt[c])
            badc += (dt[c] == -1.0).astype(jnp.int32)
        lm = pos & (badc == 0)
        accf[2] += jnp.sum(jnp.where(lm, t10, 0.0))

    @pl.when(step == GRID - 1)
    def _finalize():
        cnt_pos = acci[0]
        cnt_neg = acci[1]
        k = jnp.minimum(NEG_POS_RATIO * cnt_pos, cnt_neg)

        def count_ge(cand):
            def body(ci, c):
                blk = key_s[:, pl.ds(ci * CH, CH)]
                return c + jnp.sum(blk >= cand, dtype=jnp.int32)
            return lax.fori_loop(0, NC, body, jnp.int32(0))

        t0 = jnp.where(count_ge(jnp.int32(0)) >= k, jnp.int32(0),
                       jnp.int32(INT32_MIN))

        def bs_body(i, t):
            stp = jnp.int32(1) << (30 - i)
            cand = t + stp
            return jnp.where(count_ge(cand) >= k, cand, t)

        t = lax.fori_loop(0, 31, bs_body, t0)

        def fin_body(ci, carry):
            cg, sg, ce, se = carry
            kb = key_s[:, pl.ds(ci * CH, CH)]
            vb = nl_s[:, pl.ds(ci * CH, CH)]
            gt = kb > t
            eq = kb == t
            cg += jnp.sum(gt, dtype=jnp.int32)
            sg += jnp.sum(jnp.where(gt, vb, 0.0))
            ce += jnp.sum(eq, dtype=jnp.int32)
            se += jnp.sum(jnp.where(eq, vb, 0.0))
            return cg, sg, ce, se

        cg, sg, ce, se = lax.fori_loop(
            0, NC, fin_body,
            (jnp.int32(0), jnp.float32(0.0), jnp.int32(0), jnp.float32(0.0)))

        tval = se / jnp.maximum(ce, 1).astype(jnp.float32)
        sum_topk = jnp.where(k > 0,
                             sg + (k - cg).astype(jnp.float32) * tval,
                             0.0)

        nf = jnp.maximum(1.0, cnt_pos.astype(jnp.float32))
        v0 = (accf[0] + sum_topk) / nf
        v1 = BOX_WEIGHT * accf[1] / nf
        v2 = accf[2] / nf

        r = lax.broadcasted_iota(jnp.int32, (8, LANES), 0)
        c = lax.broadcasted_iota(jnp.int32, (8, LANES), 1)
        outv = jnp.where((r == 0) & (c == 0), v0,
                         jnp.where((r == 0) & (c == 1), v1,
                                   jnp.where((r == 0) & (c == 2), v2, 0.0)))
        out[...] = outv


def _bt(s):
    return jnp.minimum(s // NC, 3)


def _cb(s):
    return jnp.where(s == NSTEP, NC - 1, s % NC)


@jax.jit
def kernel(loc_p, conf_p, land_p, loc_t, conf_t, land_t):
    ct = conf_t.astype(jnp.int32)
    cpv = conf_p.transpose(0, 2, 1)
    lpv = loc_p.transpose(0, 2, 1)
    ltv = loc_t.transpose(0, 2, 1)
    dpv = land_p.transpose(2, 0, 1)
    dtv = land_t.transpose(2, 0, 1)

    out = pl.pallas_call(
        _mbl_kernel,
        grid=(GRID,),
        in_specs=[
            pl.BlockSpec((8, CH), lambda s: (_bt(s), _cb(s))),
            pl.BlockSpec((8, 2, CH), lambda s: (_bt(s), 0, _cb(s))),
            pl.BlockSpec((8, 4, CH), lambda s: (_bt(s), 0, _cb(s))),
            pl.BlockSpec((8, 4, CH), lambda s: (_bt(s), 0, _cb(s))),
            pl.BlockSpec((10, 8, CH), lambda s: (0, _bt(s), _cb(s))),
            pl.BlockSpec((10, 8, CH), lambda s: (0, _bt(s), _cb(s))),
        ],
        out_specs=pl.BlockSpec((8, LANES), lambda s: (0, 0)),
        out_shape=jax.ShapeDtypeStruct((8, LANES), jnp.float32),
        scratch_shapes=[
            pltpu.VMEM((B, PW), jnp.int32),
            pltpu.VMEM((B, PW), jnp.float32),
            pltpu.SMEM((4,), jnp.float32),
            pltpu.SMEM((4,), jnp.int32),
        ],
    )(ct, cpv, lpv, ltv, dpv, dtv)

    return (out[0, 0], out[0, 1], out[0, 2])


# speedup vs baseline: 1.0811x; 1.0811x over previous
import jax
import jax.numpy as jnp
from jax import lax
from jax.experimental import pallas as pl
from jax.experimental.pallas import tpu as pltpu

B, P = 32, 16800
LANES = 128
CH = 2176
NC = 8
PW = NC * CH
NSTEP = 4 * NC
GRID = NSTEP + 1
NEG_POS_RATIO = 7
BOX_WEIGHT = 2.0
INT32_MIN = -2147483648


def _smooth_l1(x):
    a = jnp.abs(x)
    return jnp.where(a < 1.0, 0.5 * x * x, a - 0.5)


def _mbl_kernel(ct, cp, lp, lt, dp, dt, out, key_s, nl_s, accv, cntv, accf):
    step = pl.program_id(0)

    @pl.when(step == 0)
    def _init():
        accv[...] = jnp.zeros((3, 8, CH), jnp.float32)
        cntv[...] = jnp.zeros((2, 8, CH), jnp.int32)

    @pl.when(step < NSTEP)
    def _accumulate():
        t = step // NC
        cb = step % NC

        lane = lax.broadcasted_iota(jnp.int32, (8, CH), 1)
        valid = cb * CH + lane < P

        labels = ct[...]
        pos = (labels > 0) & valid
        neg = (labels == 0) & valid

        x0 = cp[:, 0, :]
        x1 = cp[:, 1, :]
        d = x1 - x0
        z = jnp.where(pos, -d, d)
        spl = jnp.maximum(z, 0.0) + jnp.log(1.0 + jnp.exp(-jnp.abs(z)))

        accv[0] += jnp.where(pos, spl, 0.0)
        cntv[0] += pos.astype(jnp.int32)
        cntv[1] += neg.astype(jnp.int32)

        bits = lax.bitcast_convert_type(d, jnp.int32)
        key = jnp.where(bits >= 0, bits, INT32_MIN - bits)
        key = jnp.where(neg, key, INT32_MIN)
        key_s[pl.ds(t * 8, 8), pl.ds(cb * CH, CH)] = key
        nl_s[pl.ds(t * 8, 8), pl.ds(cb * CH, CH)] = jnp.where(neg, spl, 0.0)

        t4 = jnp.sum(_smooth_l1(lp[...] - lt[...]), axis=1)
        accv[1] += jnp.where(pos, t4, 0.0)

        dtv = dt[...]
        t10 = jnp.sum(_smooth_l1(dp[...] - dtv), axis=0)
        good = jnp.all(dtv != -1.0, axis=0)
        accv[2] += jnp.where(pos & good, t10, 0.0)

    @pl.when(step == GRID - 1)
    def _finalize():
        cnt_pos = jnp.sum(cntv[0])
        cnt_neg = jnp.sum(cntv[1])
        k = jnp.minimum(NEG_POS_RATIO * cnt_pos, cnt_neg)

        def count_ge(cand):
            def body(ci, c):
                blk = key_s[:, pl.ds(ci * CH, CH)]
                return c + jnp.sum(blk >= cand, dtype=jnp.int32)
            return lax.fori_loop(0, NC, body, jnp.int32(0))

        t0 = jnp.where(count_ge(jnp.int32(0)) >= k, jnp.int32(0),
                       jnp.int32(INT32_MIN))

        def bs_body(i, t):
            stp = jnp.int32(1) << (30 - i)
            cand = t + stp
            return jnp.where(count_ge(cand) >= k, cand, t)

        t = lax.fori_loop(0, 31, bs_body, t0)

        def fin_body(ci, carry):
            cg, sg, ce, se = carry
            kb = key_s[:, pl.ds(ci * CH, CH)]
            vb = nl_s[:, pl.ds(ci * CH, CH)]
            gt = kb > t
            eq = kb == t
            cg += jnp.sum(gt, dtype=jnp.int32)
            sg += jnp.sum(jnp.where(gt, vb, 0.0))
            ce += jnp.sum(eq, dtype=jnp.int32)
            se += jnp.sum(jnp.where(eq, vb, 0.0))
            return cg, sg, ce, se

        cg, sg, ce, se = lax.fori_loop(
            0, NC, fin_body,
            (jnp.int32(0), jnp.float32(0.0), jnp.int32(0), jnp.float32(0.0)))

        tval = se / jnp.maximum(ce, 1).astype(jnp.float32)
        sum_topk = jnp.where(k > 0,
                             sg + (k - cg).astype(jnp.float32) * tval,
                             0.0)

        nf = jnp.maximum(1.0, cnt_pos.astype(jnp.float32))
        v0 = (jnp.sum(accv[0]) + sum_topk) / nf
        v1 = BOX_WEIGHT * jnp.sum(accv[1]) / nf
        v2 = jnp.sum(accv[2]) / nf

        r = lax.broadcasted_iota(jnp.int32, (8, LANES), 0)
        c = lax.broadcasted_iota(jnp.int32, (8, LANES), 1)
        outv = jnp.where((r == 0) & (c == 0), v0,
                         jnp.where((r == 0) & (c == 1), v1,
                                   jnp.where((r == 0) & (c == 2), v2, 0.0)))
        out[...] = outv


def _bt(s):
    return jnp.minimum(s // NC, 3)


def _cb(s):
    return jnp.where(s == NSTEP, NC - 1, s % NC)


@jax.jit
def kernel(loc_p, conf_p, land_p, loc_t, conf_t, land_t):
    ct = conf_t.astype(jnp.int32)
    cpv = conf_p.transpose(0, 2, 1)
    lpv = loc_p.transpose(0, 2, 1)
    ltv = loc_t.transpose(0, 2, 1)
    dpv = land_p.transpose(2, 0, 1)
    dtv = land_t.transpose(2, 0, 1)

    out = pl.pallas_call(
        _mbl_kernel,
        grid=(GRID,),
        in_specs=[
            pl.BlockSpec((8, CH), lambda s: (_bt(s), _cb(s))),
            pl.BlockSpec((8, 2, CH), lambda s: (_bt(s), 0, _cb(s))),
            pl.BlockSpec((8, 4, CH), lambda s: (_bt(s), 0, _cb(s))),
            pl.BlockSpec((8, 4, CH), lambda s: (_bt(s), 0, _cb(s))),
            pl.BlockSpec((10, 8, CH), lambda s: (0, _bt(s), _cb(s))),
            pl.BlockSpec((10, 8, CH), lambda s: (0, _bt(s), _cb(s))),
        ],
        out_specs=pl.BlockSpec((8, LANES), lambda s: (0, 0)),
        out_shape=jax.ShapeDtypeStruct((8, LANES), jnp.float32),
        scratch_shapes=[
            pltpu.VMEM((B, PW), jnp.int32),
            pltpu.VMEM((B, PW), jnp.float32),
            pltpu.VMEM((3, 8, CH), jnp.float32),
            pltpu.VMEM((2, 8, CH), jnp.int32),
            pltpu.SMEM((4,), jnp.float32),
        ],
    )(ct, cpv, lpv, ltv, dpv, dtv)

    return (out[0, 0], out[0, 1], out[0, 2])


# speedup vs baseline: 1.3333x; 1.2332x over previous
import jax
import jax.numpy as jnp
from jax import lax
from jax.experimental import pallas as pl
from jax.experimental.pallas import tpu as pltpu

B, P = 32, 16800
LANES = 128
CH = 4352
NC = 4
PW = NC * CH
NSTEP = 4 * NC
GRID = NSTEP + 1
NEG_POS_RATIO = 7
BOX_WEIGHT = 2.0
INT32_MIN = -2147483648


def _smooth_l1(x):
    a = jnp.abs(x)
    m = jnp.minimum(a, 1.0)
    return 0.5 * m * m + (a - m)


def _mbl_kernel(ct, cp, lp, lt, dp, dt, out, key_s, nl_s, accv, cntv, accf):
    step = pl.program_id(0)

    @pl.when(step == 0)
    def _init():
        accv[...] = jnp.zeros((3, 8, CH), jnp.float32)
        cntv[...] = jnp.zeros((2, 8, CH), jnp.int32)

    @pl.when(step < NSTEP)
    def _accumulate():
        t = step // NC
        cb = step % NC

        lane = lax.broadcasted_iota(jnp.int32, (8, CH), 1)
        valid = cb * CH + lane < P

        labels = ct[...]
        pos = (labels > 0) & valid
        neg = (labels == 0) & valid

        x0 = cp[:, 0, :]
        x1 = cp[:, 1, :]
        d = x1 - x0
        z = jnp.where(pos, -d, d)
        spl = jnp.maximum(z, 0.0) + jnp.log(1.0 + jnp.exp(-jnp.abs(z)))

        accv[0] += jnp.where(pos, spl, 0.0)
        cntv[0] += pos.astype(jnp.int32)
        cntv[1] += neg.astype(jnp.int32)

        bits = lax.bitcast_convert_type(d, jnp.int32)
        key = jnp.where(bits >= 0, bits, INT32_MIN - bits)
        key = jnp.where(neg, key, INT32_MIN)
        key_s[pl.ds(t * 8, 8), pl.ds(cb * CH, CH)] = key
        nl_s[pl.ds(t * 8, 8), pl.ds(cb * CH, CH)] = jnp.where(neg, spl, 0.0)

        t4 = _smooth_l1(lp[:, 0, :] - lt[:, 0, :])
        for c in range(1, 4):
            t4 += _smooth_l1(lp[:, c, :] - lt[:, c, :])
        accv[1] += jnp.where(pos, t4, 0.0)

        t10 = _smooth_l1(dp[0] - dt[0])
        good = dt[0] != -1.0
        for c in range(1, 10):
            t10 += _smooth_l1(dp[c] - dt[c])
            good &= dt[c] != -1.0
        accv[2] += jnp.where(pos & good, t10, 0.0)

    @pl.when(step == GRID - 1)
    def _finalize():
        cnt_pos = jnp.sum(cntv[0])
        cnt_neg = jnp.sum(cntv[1])
        k = jnp.minimum(NEG_POS_RATIO * cnt_pos, cnt_neg)

        def count_ge(cand):
            def body(ci, c):
                blk = key_s[:, pl.ds(ci * CH, CH)]
                return c + jnp.sum(blk >= cand, dtype=jnp.int32)
            return lax.fori_loop(0, NC, body, jnp.int32(0))

        t0 = jnp.where(count_ge(jnp.int32(0)) >= k, jnp.int32(0),
                       jnp.int32(INT32_MIN))

        def bs_body(i, t):
            stp = jnp.int32(1) << (30 - i)
            cand = t + stp
            return jnp.where(count_ge(cand) >= k, cand, t)

        t = lax.fori_loop(0, 31, bs_body, t0)

        def fin_body(ci, carry):
            cg, sg, ce, se = carry
            kb = key_s[:, pl.ds(ci * CH, CH)]
            vb = nl_s[:, pl.ds(ci * CH, CH)]
            gt = kb > t
            eq = kb == t
            cg += jnp.sum(gt, dtype=jnp.int32)
            sg += jnp.sum(jnp.where(gt, vb, 0.0))
            ce += jnp.sum(eq, dtype=jnp.int32)
            se += jnp.sum(jnp.where(eq, vb, 0.0))
            return cg, sg, ce, se

        cg, sg, ce, se = lax.fori_loop(
            0, NC, fin_body,
            (jnp.int32(0), jnp.float32(0.0), jnp.int32(0), jnp.float32(0.0)))

        tval = se / jnp.maximum(ce, 1).astype(jnp.float32)
        sum_topk = jnp.where(k > 0,
                             sg + (k - cg).astype(jnp.float32) * tval,
                             0.0)

        nf = jnp.maximum(1.0, cnt_pos.astype(jnp.float32))
        v0 = (jnp.sum(accv[0]) + sum_topk) / nf
        v1 = BOX_WEIGHT * jnp.sum(accv[1]) / nf
        v2 = jnp.sum(accv[2]) / nf

        r = lax.broadcasted_iota(jnp.int32, (8, LANES), 0)
        c = lax.broadcasted_iota(jnp.int32, (8, LANES), 1)
        outv = jnp.where((r == 0) & (c == 0), v0,
                         jnp.where((r == 0) & (c == 1), v1,
                                   jnp.where((r == 0) & (c == 2), v2, 0.0)))
        out[...] = outv


def _bt(s):
    return jnp.minimum(s // NC, 3)


def _cb(s):
    return jnp.where(s == NSTEP, NC - 1, s % NC)


@jax.jit
def kernel(loc_p, conf_p, land_p, loc_t, conf_t, land_t):
    ct = conf_t.astype(jnp.int32)
    cpv = conf_p.transpose(0, 2, 1)
    lpv = loc_p.transpose(0, 2, 1)
    ltv = loc_t.transpose(0, 2, 1)
    dpv = land_p.transpose(2, 0, 1)
    dtv = land_t.transpose(2, 0, 1)

    out = pl.pallas_call(
        _mbl_kernel,
        grid=(GRID,),
        in_specs=[
            pl.BlockSpec((8, CH), lambda s: (_bt(s), _cb(s))),
            pl.BlockSpec((8, 2, CH), lambda s: (_bt(s), 0, _cb(s))),
            pl.BlockSpec((8, 4, CH), lambda s: (_bt(s), 0, _cb(s))),
            pl.BlockSpec((8, 4, CH), lambda s: (_bt(s), 0, _cb(s))),
            pl.BlockSpec((10, 8, CH), lambda s: (0, _bt(s), _cb(s))),
            pl.BlockSpec((10, 8, CH), lambda s: (0, _bt(s), _cb(s))),
        ],
        out_specs=pl.BlockSpec((8, LANES), lambda s: (0, 0)),
        out_shape=jax.ShapeDtypeStruct((8, LANES), jnp.float32),
        scratch_shapes=[
            pltpu.VMEM((B, PW), jnp.int32),
            pltpu.VMEM((B, PW), jnp.float32),
            pltpu.VMEM((3, 8, CH), jnp.float32),
            pltpu.VMEM((2, 8, CH), jnp.int32),
            pltpu.SMEM((4,), jnp.float32),
        ],
    )(ct, cpv, lpv, ltv, dpv, dtv)

    return (out[0, 0], out[0, 1], out[0, 2])


# speedup vs baseline: 1.5808x; 1.1857x over previous
import jax
import jax.numpy as jnp
from jax import lax
from jax.experimental import pallas as pl
from jax.experimental.pallas import tpu as pltpu

B, P = 32, 16800
LANES = 128
CH = 4352
NC = 4
PW = NC * CH
NSTEP = 4 * NC
GRID = NSTEP + 1
NEG_POS_RATIO = 7
BOX_WEIGHT = 2.0
INT32_MIN = -2147483648


def _smooth_l1(x):
    a = jnp.abs(x)
    m = jnp.minimum(a, 1.0)
    return 0.5 * m * m + (a - m)


def _mbl_kernel(ct, cp, lp, lt, dp, dt, out, key_s, nl_s, accv, cntv, accf):
    step = pl.program_id(0)

    @pl.when(step == 0)
    def _init():
        accv[...] = jnp.zeros((3, 8, CH), jnp.float32)
        cntv[...] = jnp.zeros((2, 8, CH), jnp.int32)

    @pl.when(step < NSTEP)
    def _accumulate():
        t = step // NC
        cb = step % NC

        lane = lax.broadcasted_iota(jnp.int32, (8, CH), 1)
        valid = cb * CH + lane < P

        labels = ct[...]
        pos = (labels > 0) & valid
        neg = (labels == 0) & valid

        x0 = cp[:, 0, :]
        x1 = cp[:, 1, :]
        d = x1 - x0
        z = jnp.where(pos, -d, d)
        spl = jnp.maximum(z, 0.0) + jnp.log(1.0 + jnp.exp(-jnp.abs(z)))

        accv[0] += jnp.where(pos, spl, 0.0)
        cntv[0] += pos.astype(jnp.int32)
        cntv[1] += neg.astype(jnp.int32)

        bits = lax.bitcast_convert_type(d, jnp.int32)
        key = jnp.where(bits >= 0, bits, INT32_MIN - bits)
        key = jnp.where(neg, key, INT32_MIN)
        key_s[pl.ds(t * 8, 8), pl.ds(cb * CH, CH)] = key
        nl_s[pl.ds(t * 8, 8), pl.ds(cb * CH, CH)] = jnp.where(neg, spl, 0.0)

        t4 = _smooth_l1(lp[:, 0, :] - lt[:, 0, :])
        for c in range(1, 4):
            t4 += _smooth_l1(lp[:, c, :] - lt[:, c, :])
        accv[1] += jnp.where(pos, t4, 0.0)

        t10 = _smooth_l1(dp[0] - dt[0])
        good = dt[0] != -1.0
        for c in range(1, 10):
            t10 += _smooth_l1(dp[c] - dt[c])
            good &= dt[c] != -1.0
        accv[2] += jnp.where(pos & good, t10, 0.0)

    @pl.when(step == GRID - 1)
    def _finalize():
        cnt_pos = jnp.sum(cntv[0])
        cnt_neg = jnp.sum(cntv[1])
        k = jnp.minimum(NEG_POS_RATIO * cnt_pos, cnt_neg)

        def count_ge(cand):
            def body(ci, c):
                blk = key_s[:, pl.ds(ci * CH, CH)]
                return c + jnp.sum(blk >= cand, dtype=jnp.int32)
            return lax.fori_loop(0, NC, body, jnp.int32(0))

        def count3(c1, c2, c3):
            def body(ci, carry):
                a1, a2, a3 = carry
                blk = key_s[:, pl.ds(ci * CH, CH)]
                a1 += jnp.sum(blk >= c1, dtype=jnp.int32)
                a2 += jnp.sum(blk >= c2, dtype=jnp.int32)
                a3 += jnp.sum(blk >= c3, dtype=jnp.int32)
                return a1, a2, a3
            z = jnp.int32(0)
            return lax.fori_loop(0, NC, body, (z, z, z))

        t0 = jnp.where(count_ge(jnp.int32(0)) >= k, jnp.int32(0),
                       jnp.int32(INT32_MIN))

        def r4_body(i, t):
            s = jnp.int32(1) << (29 - 2 * i)
            c1, c2, c3 = t + s, t + 2 * s, t + 3 * s
            n1, n2, n3 = count3(c1, c2, c3)
            t = jnp.where(n1 >= k, c1, t)
            t = jnp.where(n2 >= k, c2, t)
            t = jnp.where(n3 >= k, c3, t)
            return t

        t = lax.fori_loop(0, 15, r4_body, t0)
        t = jnp.where(count_ge(t + 1) >= k, t + 1, t)

        def fin_body(ci, carry):
            cg, sg, ce, se = carry
            kb = key_s[:, pl.ds(ci * CH, CH)]
            vb = nl_s[:, pl.ds(ci * CH, CH)]
            gt = kb > t
            eq = kb == t
            cg += jnp.sum(gt, dtype=jnp.int32)
            sg += jnp.sum(jnp.where(gt, vb, 0.0))
            ce += jnp.sum(eq, dtype=jnp.int32)
            se += jnp.sum(jnp.where(eq, vb, 0.0))
            return cg, sg, ce, se

        cg, sg, ce, se = lax.fori_loop(
            0, NC, fin_body,
            (jnp.int32(0), jnp.float32(0.0), jnp.int32(0), jnp.float32(0.0)))

        tval = se / jnp.maximum(ce, 1).astype(jnp.float32)
        sum_topk = jnp.where(k > 0,
                             sg + (k - cg).astype(jnp.float32) * tval,
                             0.0)

        nf = jnp.maximum(1.0, cnt_pos.astype(jnp.float32))
        v0 = (jnp.sum(accv[0]) + sum_topk) / nf
        v1 = BOX_WEIGHT * jnp.sum(accv[1]) / nf
        v2 = jnp.sum(accv[2]) / nf

        r = lax.broadcasted_iota(jnp.int32, (8, LANES), 0)
        c = lax.broadcasted_iota(jnp.int32, (8, LANES), 1)
        outv = jnp.where((r == 0) & (c == 0), v0,
                         jnp.where((r == 0) & (c == 1), v1,
                                   jnp.where((r == 0) & (c == 2), v2, 0.0)))
        out[...] = outv


def _bt(s):
    return jnp.minimum(s // NC, 3)


def _cb(s):
    return jnp.where(s == NSTEP, NC - 1, s % NC)


@jax.jit
def kernel(loc_p, conf_p, land_p, loc_t, conf_t, land_t):
    ct = conf_t.astype(jnp.int32)
    cpv = conf_p.transpose(0, 2, 1)
    lpv = loc_p.transpose(0, 2, 1)
    ltv = loc_t.transpose(0, 2, 1)
    dpv = land_p.transpose(2, 0, 1)
    dtv = land_t.transpose(2, 0, 1)

    out = pl.pallas_call(
        _mbl_kernel,
        grid=(GRID,),
        in_specs=[
            pl.BlockSpec((8, CH), lambda s: (_bt(s), _cb(s))),
            pl.BlockSpec((8, 2, CH), lambda s: (_bt(s), 0, _cb(s))),
            pl.BlockSpec((8, 4, CH), lambda s: (_bt(s), 0, _cb(s))),
            pl.BlockSpec((8, 4, CH), lambda s: (_bt(s), 0, _cb(s))),
            pl.BlockSpec((10, 8, CH), lambda s: (0, _bt(s), _cb(s))),
            pl.BlockSpec((10, 8, CH), lambda s: (0, _bt(s), _cb(s))),
        ],
        out_specs=pl.BlockSpec((8, LANES), lambda s: (0, 0)),
        out_shape=jax.ShapeDtypeStruct((8, LANES), jnp.float32),
        scratch_shapes=[
            pltpu.VMEM((B, PW), jnp.int32),
            pltpu.VMEM((B, PW), jnp.float32),
            pltpu.VMEM((3, 8, CH), jnp.float32),
            pltpu.VMEM((2, 8, CH), jnp.int32),
            pltpu.SMEM((4,), jnp.float32),
        ],
    )(ct, cpv, lpv, ltv, dpv, dtv)

    return (out[0, 0], out[0, 1], out[0, 2])


# speedup vs baseline: 1.8882x; 1.1944x over previous
import jax
import jax.numpy as jnp
from jax import lax
from jax.experimental import pallas as pl
from jax.experimental.pallas import tpu as pltpu

B, P = 32, 16800
LANES = 128
CH = 8704
NC = 2
PW = NC * CH
NSTEP = 4 * NC
GRID = NSTEP + 1
NEG_POS_RATIO = 7
BOX_WEIGHT = 2.0
INT32_MIN = -2147483648


def _smooth_l1(x):
    a = jnp.abs(x)
    m = jnp.minimum(a, 1.0)
    return 0.5 * m * m + (a - m)


def _mbl_kernel(ct, cp, lp, lt, dp, dt, out, key_s, nl_s, accv, cntv, accf):
    step = pl.program_id(0)

    @pl.when(step == 0)
    def _init():
        accv[...] = jnp.zeros((3, 8, CH), jnp.float32)
        cntv[...] = jnp.zeros((2, 8, CH), jnp.int32)

    @pl.when(step < NSTEP)
    def _accumulate():
        t = step // NC
        cb = step % NC

        lane = lax.broadcasted_iota(jnp.int32, (8, CH), 1)
        valid = cb * CH + lane < P

        labels = ct[...]
        pos = (labels > 0) & valid
        neg = (labels == 0) & valid

        x0 = cp[:, 0, :]
        x1 = cp[:, 1, :]
        d = x1 - x0
        z = jnp.where(pos, -d, d)
        spl = jnp.maximum(z, 0.0) + jnp.log(1.0 + jnp.exp(-jnp.abs(z)))

        accv[0] += jnp.where(pos, spl, 0.0)
        cntv[0] += pos.astype(jnp.int32)
        cntv[1] += neg.astype(jnp.int32)

        bits = lax.bitcast_convert_type(d, jnp.int32)
        key = jnp.where(bits >= 0, bits, INT32_MIN - bits)
        key = jnp.where(neg, key, INT32_MIN)
        key_s[pl.ds(t * 8, 8), pl.ds(cb * CH, CH)] = key
        nl_s[pl.ds(t * 8, 8), pl.ds(cb * CH, CH)] = jnp.where(neg, spl, 0.0)

        t4 = _smooth_l1(lp[:, 0, :] - lt[:, 0, :])
        for c in range(1, 4):
            t4 += _smooth_l1(lp[:, c, :] - lt[:, c, :])
        accv[1] += jnp.where(pos, t4, 0.0)

        t10 = _smooth_l1(dp[0] - dt[0])
        good = dt[0] != -1.0
        for c in range(1, 10):
            t10 += _smooth_l1(dp[c] - dt[c])
            good &= dt[c] != -1.0
        accv[2] += jnp.where(pos & good, t10, 0.0)

    @pl.when(step == GRID - 1)
    def _finalize():
        cnt_pos = jnp.sum(cntv[0])
        cnt_neg = jnp.sum(cntv[1])
        k = jnp.minimum(NEG_POS_RATIO * cnt_pos, cnt_neg)

        def count_ge(cand):
            def body(ci, c):
                blk = key_s[:, pl.ds(ci * CH, CH)]
                return c + jnp.sum(blk >= cand, dtype=jnp.int32)
            return lax.fori_loop(0, NC, body, jnp.int32(0))

        def count3(c1, c2, c3):
            def body(ci, carry):
                a1, a2, a3 = carry
                blk = key_s[:, pl.ds(ci * CH, CH)]
                a1 += jnp.sum(blk >= c1, dtype=jnp.int32)
                a2 += jnp.sum(blk >= c2, dtype=jnp.int32)
                a3 += jnp.sum(blk >= c3, dtype=jnp.int32)
                return a1, a2, a3
            z = jnp.int32(0)
            return lax.fori_loop(0, NC, body, (z, z, z))

        t0 = jnp.where(count_ge(jnp.int32(0)) >= k, jnp.int32(0),
                       jnp.int32(INT32_MIN))

        def r4_body(i, t):
            s = jnp.int32(1) << (29 - 2 * i)
            c1, c2, c3 = t + s, t + 2 * s, t + 3 * s
            n1, n2, n3 = count3(c1, c2, c3)
            t = jnp.where(n1 >= k, c1, t)
            t = jnp.where(n2 >= k, c2, t)
            t = jnp.where(n3 >= k, c3, t)
            return t

        t = lax.fori_loop(0, 15, r4_body, t0)
        t = jnp.where(count_ge(t + 1) >= k, t + 1, t)

        def fin_body(ci, carry):
            cg, sg, ce, se = carry
            kb = key_s[:, pl.ds(ci * CH, CH)]
            vb = nl_s[:, pl.ds(ci * CH, CH)]
            gt = kb > t
            eq = kb == t
            cg += jnp.sum(gt, dtype=jnp.int32)
            sg += jnp.sum(jnp.where(gt, vb, 0.0))
            ce += jnp.sum(eq, dtype=jnp.int32)
            se += jnp.sum(jnp.where(eq, vb, 0.0))
            return cg, sg, ce, se

        cg, sg, ce, se = lax.fori_loop(
            0, NC, fin_body,
            (jnp.int32(0), jnp.float32(0.0), jnp.int32(0), jnp.float32(0.0)))

        tval = se / jnp.maximum(ce, 1).astype(jnp.float32)
        sum_topk = jnp.where(k > 0,
                             sg + (k - cg).astype(jnp.float32) * tval,
                             0.0)

        nf = jnp.maximum(1.0, cnt_pos.astype(jnp.float32))
        v0 = (jnp.sum(accv[0]) + sum_topk) / nf
        v1 = BOX_WEIGHT * jnp.sum(accv[1]) / nf
        v2 = jnp.sum(accv[2]) / nf

        r = lax.broadcasted_iota(jnp.int32, (8, LANES), 0)
        c = lax.broadcasted_iota(jnp.int32, (8, LANES), 1)
        outv = jnp.where((r == 0) & (c == 0), v0,
                         jnp.where((r == 0) & (c == 1), v1,
                                   jnp.where((r == 0) & (c == 2), v2, 0.0)))
        out[...] = outv


def _bt(s):
    return jnp.minimum(s // NC, 3)


def _cb(s):
    return jnp.where(s == NSTEP, NC - 1, s % NC)


@jax.jit
def kernel(loc_p, conf_p, land_p, loc_t, conf_t, land_t):
    ct = conf_t.astype(jnp.int32)
    cpv = conf_p.transpose(0, 2, 1)
    lpv = loc_p.transpose(0, 2, 1)
    ltv = loc_t.transpose(0, 2, 1)
    dpv = land_p.transpose(2, 0, 1)
    dtv = land_t.transpose(2, 0, 1)

    out = pl.pallas_call(
        _mbl_kernel,
        grid=(GRID,),
        in_specs=[
            pl.BlockSpec((8, CH), lambda s: (_bt(s), _cb(s))),
            pl.BlockSpec((8, 2, CH), lambda s: (_bt(s), 0, _cb(s))),
            pl.BlockSpec((8, 4, CH), lambda s: (_bt(s), 0, _cb(s))),
            pl.BlockSpec((8, 4, CH), lambda s: (_bt(s), 0, _cb(s))),
            pl.BlockSpec((10, 8, CH), lambda s: (0, _bt(s), _cb(s))),
            pl.BlockSpec((10, 8, CH), lambda s: (0, _bt(s), _cb(s))),
        ],
        out_specs=pl.BlockSpec((8, LANES), lambda s: (0, 0)),
        out_shape=jax.ShapeDtypeStruct((8, LANES), jnp.float32),
        scratch_shapes=[
            pltpu.VMEM((B, PW), jnp.int32),
            pltpu.VMEM((B, PW), jnp.float32),
            pltpu.VMEM((3, 8, CH), jnp.float32),
            pltpu.VMEM((2, 8, CH), jnp.int32),
            pltpu.SMEM((4,), jnp.float32),
        ],
    )(ct, cpv, lpv, ltv, dpv, dtv)

    return (out[0, 0], out[0, 1], out[0, 2])


# speedup vs baseline: 2.1260x; 1.1259x over previous
import jax
import jax.numpy as jnp
from jax import lax
from jax.experimental import pallas as pl
from jax.experimental.pallas import tpu as pltpu

B, P = 32, 16800
LANES = 128
CH = 16896
NC = 1
PW = NC * CH
NSTEP = 4 * NC
GRID = NSTEP + 1
NEG_POS_RATIO = 7
BOX_WEIGHT = 2.0
INT32_MIN = -2147483648


def _smooth_l1(x):
    a = jnp.abs(x)
    m = jnp.minimum(a, 1.0)
    return 0.5 * m * m + (a - m)


def _mbl_kernel(ct, cp, lp, lt, dp, dt, out, key_s, nl_s, accv, cntv, accf):
    step = pl.program_id(0)

    @pl.when(step == 0)
    def _init():
        accv[...] = jnp.zeros((3, 8, CH), jnp.float32)
        cntv[...] = jnp.zeros((2, 8, CH), jnp.int32)

    @pl.when(step < NSTEP)
    def _accumulate():
        t = step // NC
        cb = step % NC

        lane = lax.broadcasted_iota(jnp.int32, (8, CH), 1)
        valid = cb * CH + lane < P

        labels = ct[...]
        pos = (labels > 0) & valid
        neg = (labels == 0) & valid

        x0 = cp[:, 0, :]
        x1 = cp[:, 1, :]
        d = x1 - x0
        z = jnp.where(pos, -d, d)
        spl = jnp.maximum(z, 0.0) + jnp.log(1.0 + jnp.exp(-jnp.abs(z)))

        accv[0] += jnp.where(pos, spl, 0.0)
        cntv[0] += pos.astype(jnp.int32)
        cntv[1] += neg.astype(jnp.int32)

        bits = lax.bitcast_convert_type(d, jnp.int32)
        key = jnp.where(bits >= 0, bits, INT32_MIN - bits)
        key = jnp.where(neg, key, INT32_MIN)
        key_s[pl.ds(t * 8, 8), pl.ds(cb * CH, CH)] = key
        nl_s[pl.ds(t * 8, 8), pl.ds(cb * CH, CH)] = jnp.where(neg, spl, 0.0)

        t4 = _smooth_l1(lp[:, 0, :] - lt[:, 0, :])
        for c in range(1, 4):
            t4 += _smooth_l1(lp[:, c, :] - lt[:, c, :])
        accv[1] += jnp.where(pos, t4, 0.0)

        t10 = _smooth_l1(dp[0] - dt[0])
        good = dt[0] != -1.0
        for c in range(1, 10):
            t10 += _smooth_l1(dp[c] - dt[c])
            good &= dt[c] != -1.0
        accv[2] += jnp.where(pos & good, t10, 0.0)

    @pl.when(step == GRID - 1)
    def _finalize():
        cnt_pos = jnp.sum(cntv[0])
        cnt_neg = jnp.sum(cntv[1])
        k = jnp.minimum(NEG_POS_RATIO * cnt_pos, cnt_neg)

        def count_ge(cand):
            def body(ci, c):
                blk = key_s[:, pl.ds(ci * CH, CH)]
                return c + jnp.sum(blk >= cand, dtype=jnp.int32)
            return lax.fori_loop(0, NC, body, jnp.int32(0))

        def count3(c1, c2, c3):
            def body(ci, carry):
                a1, a2, a3 = carry
                blk = key_s[:, pl.ds(ci * CH, CH)]
                a1 += jnp.sum(blk >= c1, dtype=jnp.int32)
                a2 += jnp.sum(blk >= c2, dtype=jnp.int32)
                a3 += jnp.sum(blk >= c3, dtype=jnp.int32)
                return a1, a2, a3
            z = jnp.int32(0)
            return lax.fori_loop(0, NC, body, (z, z, z))

        t0 = jnp.where(count_ge(jnp.int32(0)) >= k, jnp.int32(0),
                       jnp.int32(INT32_MIN))

        def r4_body(i, t):
            s = jnp.int32(1) << (29 - 2 * i)
            c1, c2, c3 = t + s, t + 2 * s, t + 3 * s
            n1, n2, n3 = count3(c1, c2, c3)
            t = jnp.where(n1 >= k, c1, t)
            t = jnp.where(n2 >= k, c2, t)
            t = jnp.where(n3 >= k, c3, t)
            return t

        t = lax.fori_loop(0, 15, r4_body, t0)
        t = jnp.where(count_ge(t + 1) >= k, t + 1, t)

        def fin_body(ci, carry):
            cg, sg, ce, se = carry
            kb = key_s[:, pl.ds(ci * CH, CH)]
            vb = nl_s[:, pl.ds(ci * CH, CH)]
            gt = kb > t
            eq = kb == t
            cg += jnp.sum(gt, dtype=jnp.int32)
            sg += jnp.sum(jnp.where(gt, vb, 0.0))
            ce += jnp.sum(eq, dtype=jnp.int32)
            se += jnp.sum(jnp.where(eq, vb, 0.0))
            return cg, sg, ce, se

        cg, sg, ce, se = lax.fori_loop(
            0, NC, fin_body,
            (jnp.int32(0), jnp.float32(0.0), jnp.int32(0), jnp.float32(0.0)))

        tval = se / jnp.maximum(ce, 1).astype(jnp.float32)
        sum_topk = jnp.where(k > 0,
                             sg + (k - cg).astype(jnp.float32) * tval,
                             0.0)

        nf = jnp.maximum(1.0, cnt_pos.astype(jnp.float32))
        v0 = (jnp.sum(accv[0]) + sum_topk) / nf
        v1 = BOX_WEIGHT * jnp.sum(accv[1]) / nf
        v2 = jnp.sum(accv[2]) / nf

        r = lax.broadcasted_iota(jnp.int32, (8, LANES), 0)
        c = lax.broadcasted_iota(jnp.int32, (8, LANES), 1)
        outv = jnp.where((r == 0) & (c == 0), v0,
                         jnp.where((r == 0) & (c == 1), v1,
                                   jnp.where((r == 0) & (c == 2), v2, 0.0)))
        out[...] = outv


def _bt(s):
    return jnp.minimum(s // NC, 3)


def _cb(s):
    return jnp.where(s == NSTEP, NC - 1, s % NC)


@jax.jit
def kernel(loc_p, conf_p, land_p, loc_t, conf_t, land_t):
    ct = conf_t.astype(jnp.int32)
    cpv = conf_p.transpose(0, 2, 1)
    lpv = loc_p.transpose(0, 2, 1)
    ltv = loc_t.transpose(0, 2, 1)
    dpv = land_p.transpose(2, 0, 1)
    dtv = land_t.transpose(2, 0, 1)

    out = pl.pallas_call(
        _mbl_kernel,
        grid=(GRID,),
        in_specs=[
            pl.BlockSpec((8, CH), lambda s: (_bt(s), _cb(s))),
            pl.BlockSpec((8, 2, CH), lambda s: (_bt(s), 0, _cb(s))),
            pl.BlockSpec((8, 4, CH), lambda s: (_bt(s), 0, _cb(s))),
            pl.BlockSpec((8, 4, CH), lambda s: (_bt(s), 0, _cb(s))),
            pl.BlockSpec((10, 8, CH), lambda s: (0, _bt(s), _cb(s))),
            pl.BlockSpec((10, 8, CH), lambda s: (0, _bt(s), _cb(s))),
        ],
        out_specs=pl.BlockSpec((8, LANES), lambda s: (0, 0)),
        out_shape=jax.ShapeDtypeStruct((8, LANES), jnp.float32),
        scratch_shapes=[
            pltpu.VMEM((B, PW), jnp.int32),
            pltpu.VMEM((B, PW), jnp.float32),
            pltpu.VMEM((3, 8, CH), jnp.float32),
            pltpu.VMEM((2, 8, CH), jnp.int32),
            pltpu.SMEM((4,), jnp.float32),
        ],
    )(ct, cpv, lpv, ltv, dpv, dtv)

    return (out[0, 0], out[0, 1], out[0, 2])
